# baseline (speedup 1.0000x reference)
import functools

import jax
import jax.numpy as jnp
from jax.experimental import pallas as pl

_NUM_EXPERTS = 8
_TILE = 512


def _moe_body(x_ref, wg_ref, bg_ref, we_ref, be_ref, o_ref):
    xt = x_ref[...]
    logits = jnp.dot(xt, wg_ref[...], preferred_element_type=jnp.float32)
    logits = logits + bg_ref[...]
    g = jax.nn.softmax(logits, axis=-1)
    m1 = jnp.max(g, axis=-1, keepdims=True)
    g_no_top1 = jnp.where(g >= m1, -jnp.inf, g)
    m2 = jnp.max(g_no_top1, axis=-1, keepdims=True)
    wfull = jnp.where(g >= m2, g, 0.0) / (m1 + m2)
    acc = jnp.dot(wfull, be_ref[...], preferred_element_type=jnp.float32)
    xb = xt.astype(jnp.bfloat16)
    for e in range(_NUM_EXPERTS):
        ye = jnp.dot(xb, we_ref[e], preferred_element_type=jnp.float32)
        acc = acc + wfull[:, e:e + 1] * ye
    o_ref[...] = acc


def _forward(x, Wg, bg, We, be, *, interpret=False):
    B, S, D = x.shape
    E = Wg.shape[-1]
    n = B * S
    xf = x.reshape(n, D)
    grid = (n // _TILE,)
    out = pl.pallas_call(
        _moe_body,
        grid=grid,
        in_specs=[
            pl.BlockSpec((_TILE, D), lambda i: (i, 0)),
            pl.BlockSpec((D, E), lambda i: (0, 0)),
            pl.BlockSpec((1, E), lambda i: (0, 0)),
            pl.BlockSpec((E, D, D), lambda i: (0, 0, 0)),
            pl.BlockSpec((E, D), lambda i: (0, 0)),
        ],
        out_specs=pl.BlockSpec((_TILE, D), lambda i: (i, 0)),
        out_shape=jax.ShapeDtypeStruct((n, D), jnp.float32),
        interpret=interpret,
    )(xf, Wg, bg.reshape(1, E), We.astype(jnp.bfloat16), be)
    return out.reshape(B, S, D)


def kernel(x, Wg, bg, We, be):
    return _forward(x, Wg, bg, We, be)
